# baseline (speedup 1.0000x reference)
import math

import jax
import jax.numpy as jnp
from jax import lax
from jax.experimental import pallas as pl
from jax.experimental.pallas import tpu as pltpu
from jax.experimental.pallas import tpu_sc as plsc

N_LEVELS = 16
FDIM = 2
TSZ = 1 << 19
BASE_RES = 16
PER_LEVEL_SCALE = 1.5
P1 = 2654435761
P2 = 805459861

NC = 2
NS = 16
NW = NC * NS
LANES = 16

PGRP = TSZ // 8
RES = [int(math.floor(BASE_RES * PER_LEVEL_SCALE ** l)) for l in range(N_LEVELS)]
DENSE = [(r + 1) ** 3 <= TSZ for r in RES]
CORNERS = [(cx, cy, cz) for cz in (0, 1) for cy in (0, 1) for cx in (0, 1)]

C = 256
UNROLL = 1


def _body(xn, tab, out, coords_v, frac_v, *rest):
    grp_vs = (rest[0:8], rest[8:16])
    off_vs = (rest[16:24], rest[24:32])
    rows0_vs = (rest[32:40], rest[40:48])
    rows1_vs = (rest[48:56], rest[56:64])
    obuf_v = rest[64]
    sems = (rest[65], rest[66])
    n = out.shape[0]
    ppt = n // NW
    nchunk = ppt // C
    wid = lax.axis_index("s") * NC + lax.axis_index("c")
    lane = jnp.arange(LANES, dtype=jnp.int32)
    zeros16 = jnp.zeros((LANES,), jnp.int32)
    ones16 = jnp.full((LANES,), 1, jnp.int32)
    twos16 = jnp.full((LANES,), 2, jnp.int32)

    def a_and_fire(l, par):
        res = RES[l]
        f0 = 3 * par

        @plsc.parallel_loop(0, C // LANES, unroll=UNROLL)
        def pa(i, l=l, res=res, par=par, f0=f0):
            o = i * LANES
            rows16 = o + lane
            x = plsc.load_gather(coords_v, [rows16, zeros16])
            y = plsc.load_gather(coords_v, [rows16, ones16])
            z = plsc.load_gather(coords_v, [rows16, twos16])
            px = x * float(res)
            py = y * float(res)
            pz = z * float(res)
            ix = px.astype(jnp.int32)
            iy = py.astype(jnp.int32)
            iz = pz.astype(jnp.int32)
            plsc.store_scatter(frac_v, [jnp.full((LANES,), f0, jnp.int32), rows16],
                               px - ix.astype(jnp.float32))
            plsc.store_scatter(frac_v, [jnp.full((LANES,), f0 + 1, jnp.int32), rows16],
                               py - iy.astype(jnp.float32))
            plsc.store_scatter(frac_v, [jnp.full((LANES,), f0 + 2, jnp.int32), rows16],
                               pz - iz.astype(jnp.float32))
            if DENSE[l]:
                s = res + 1
                base_d = ix + s * (iy + s * iz)
                for ci, (cx, cy, cz) in enumerate(CORNERS):
                    idx = base_d + (cx + s * cy + s * s * cz)
                    plsc.store_scatter(grp_vs[par][ci], [rows16], idx >> 3)
                    plsc.store_scatter(off_vs[par][ci], [rows16], idx & 7)
            else:
                ux = ix.astype(jnp.uint32)
                uy = iy.astype(jnp.uint32) * jnp.uint32(P1)
                uz = iz.astype(jnp.uint32) * jnp.uint32(P2)
                ux1 = ux + jnp.uint32(1)
                uy1 = uy + jnp.uint32(P1)
                uz1 = uz + jnp.uint32(P2)
                for ci, (cx, cy, cz) in enumerate(CORNERS):
                    h = (ux1 if cx else ux) ^ (uy1 if cy else uy) ^ (uz1 if cz else uz)
                    idx = (h & jnp.uint32(TSZ - 1)).astype(jnp.int32)
                    plsc.store_scatter(grp_vs[par][ci], [rows16], idx >> 3)
                    plsc.store_scatter(off_vs[par][ci], [rows16], idx & 7)

        tab0 = tab.at[pl.ds((2 * l) * PGRP, PGRP)]
        tab1 = tab.at[pl.ds((2 * l + 1) * PGRP, PGRP)]
        copies = []
        for ci in range(8):
            copies.append(pltpu.async_copy(
                tab0.at[grp_vs[par][ci]], rows0_vs[par][ci], sems[par]))
            copies.append(pltpu.async_copy(
                tab1.at[grp_vs[par][ci]], rows1_vs[par][ci], sems[par]))
        return copies

    def interp(l, par):
        f0 = 3 * par

        @plsc.parallel_loop(0, C // LANES, unroll=UNROLL)
        def pc(i, l=l, par=par, f0=f0):
            o = i * LANES
            rows16 = o + lane
            fx = plsc.load_gather(frac_v, [jnp.full((LANES,), f0, jnp.int32), rows16])
            fy = plsc.load_gather(frac_v, [jnp.full((LANES,), f0 + 1, jnp.int32), rows16])
            fz = plsc.load_gather(frac_v, [jnp.full((LANES,), f0 + 2, jnp.int32), rows16])
            gx = 1.0 - fx
            gy = 1.0 - fy
            gz = 1.0 - fz
            wxy = {(0, 0): gx * gy, (1, 0): fx * gy,
                   (0, 1): gx * fy, (1, 1): fx * fy}
            acc0 = jnp.zeros((LANES,), jnp.float32)
            acc1 = jnp.zeros((LANES,), jnp.float32)
            for ci, (cx, cy, cz) in enumerate(CORNERS):
                w = wxy[(cx, cy)] * (fz if cz else gz)
                lo = plsc.load_gather(off_vs[par][ci], [rows16])
                v0 = plsc.load_gather(rows0_vs[par][ci], [rows16, lo])
                v1 = plsc.load_gather(rows1_vs[par][ci], [rows16, lo])
                acc0 = acc0 + v0 * w
                acc1 = acc1 + v1 * w
            plsc.store_scatter(
                obuf_v, [rows16, jnp.full((LANES,), 2 * l, jnp.int32)], acc0)
            plsc.store_scatter(
                obuf_v, [rows16, jnp.full((LANES,), 2 * l + 1, jnp.int32)], acc1)

    def chunk_body(g, carry):
        base = wid * ppt + g * C
        pltpu.sync_copy(xn.at[pl.ds(base, C)], coords_v)

        copies = a_and_fire(0, 0)
        for l in range(N_LEVELS):
            par = l % 2
            nxt = a_and_fire(l + 1, 1 - par) if l + 1 < N_LEVELS else None
            for cp in copies:
                cp.wait()
            interp(l, par)
            copies = nxt

        pltpu.sync_copy(obuf_v, out.at[pl.ds(base, C)])
        return carry

    lax.fori_loop(0, nchunk, chunk_body, 0)


def kernel(xc, table, bbox_min, bbox_max):
    n = xc.shape[0]
    assert n % (NW * C) == 0
    scale = jnp.clip(bbox_max - bbox_min, 1e-6, None)
    xn = (xc - bbox_min[None, :]) / scale[None, :]
    tab = jnp.swapaxes(table, 1, 2).reshape(N_LEVELS * FDIM * TSZ // 8, 8)

    mesh = plsc.VectorSubcoreMesh(core_axis_name="c", subcore_axis_name="s")
    f = pl.kernel(
        _body,
        out_type=jax.ShapeDtypeStruct((n, N_LEVELS * FDIM), jnp.float32),
        mesh=mesh,
        compiler_params=pltpu.CompilerParams(
            needs_layout_passes=False, use_tc_tiling_on_sc=False),
        scratch_types=[
            pltpu.VMEM((C, 3), jnp.float32),
            pltpu.VMEM((6, C), jnp.float32),
            *[pltpu.VMEM((C,), jnp.int32) for _ in range(16)],
            *[pltpu.VMEM((C,), jnp.int32) for _ in range(16)],
            *[pltpu.VMEM((C, 8), jnp.float32) for _ in range(16)],
            *[pltpu.VMEM((C, 8), jnp.float32) for _ in range(16)],
            pltpu.VMEM((C, N_LEVELS * FDIM), jnp.float32),
            pltpu.SemaphoreType.DMA,
            pltpu.SemaphoreType.DMA,
        ],
    )
    return f(xn, tab)


# speedup vs baseline: 1.6413x; 1.6413x over previous
import math

import jax
import jax.numpy as jnp
from jax import lax
from jax.experimental import pallas as pl
from jax.experimental.pallas import tpu as pltpu
from jax.experimental.pallas import tpu_sc as plsc

N_LEVELS = 16
FDIM = 2
TSZ = 1 << 19
BASE_RES = 16
PER_LEVEL_SCALE = 1.5
P1 = 2654435761
P2 = 805459861

NC = 2
NS = 16
NW = NC * NS
LANES = 16

GRP = TSZ // 4
RES = [int(math.floor(BASE_RES * PER_LEVEL_SCALE ** l)) for l in range(N_LEVELS)]
DENSE = [(r + 1) ** 3 <= TSZ for r in RES]
CORNERS = [(cx, cy, cz) for cz in (0, 1) for cy in (0, 1) for cx in (0, 1)]

C = 512
UNROLL = 2


def _body(xn, tab, out, coords_v, frac_v, *rest):
    grp_vs = (rest[0:8], rest[8:16])
    off_vs = (rest[16:24], rest[24:32])
    rows_vs = (rest[32:40], rest[40:48])
    obuf_v = rest[48]
    sems = (rest[49], rest[50])
    n = out.shape[0]
    ppt = n // NW
    nchunk = ppt // C
    wid = lax.axis_index("s") * NC + lax.axis_index("c")
    lane = jnp.arange(LANES, dtype=jnp.int32)
    zeros16 = jnp.zeros((LANES,), jnp.int32)
    ones16 = jnp.full((LANES,), 1, jnp.int32)
    twos16 = jnp.full((LANES,), 2, jnp.int32)

    def a_and_fire(l, par):
        res = RES[l]
        f0 = 3 * par

        @plsc.parallel_loop(0, C // LANES, unroll=UNROLL)
        def pa(i, l=l, res=res, par=par, f0=f0):
            o = i * LANES
            rows16 = o + lane
            x = plsc.load_gather(coords_v, [rows16, zeros16])
            y = plsc.load_gather(coords_v, [rows16, ones16])
            z = plsc.load_gather(coords_v, [rows16, twos16])
            px = x * float(res)
            py = y * float(res)
            pz = z * float(res)
            ix = px.astype(jnp.int32)
            iy = py.astype(jnp.int32)
            iz = pz.astype(jnp.int32)
            plsc.store_scatter(frac_v, [jnp.full((LANES,), f0, jnp.int32), rows16],
                               px - ix.astype(jnp.float32))
            plsc.store_scatter(frac_v, [jnp.full((LANES,), f0 + 1, jnp.int32), rows16],
                               py - iy.astype(jnp.float32))
            plsc.store_scatter(frac_v, [jnp.full((LANES,), f0 + 2, jnp.int32), rows16],
                               pz - iz.astype(jnp.float32))
            if DENSE[l]:
                s = res + 1
                base_d = ix + s * (iy + s * iz)
                for ci, (cx, cy, cz) in enumerate(CORNERS):
                    idx = base_d + (cx + s * cy + s * s * cz)
                    plsc.store_scatter(grp_vs[par][ci], [rows16],
                                       (idx >> 2) + l * GRP)
                    plsc.store_scatter(off_vs[par][ci], [rows16], (idx & 3) << 1)
            else:
                ux = ix.astype(jnp.uint32)
                uy = iy.astype(jnp.uint32) * jnp.uint32(P1)
                uz = iz.astype(jnp.uint32) * jnp.uint32(P2)
                ux1 = ux + jnp.uint32(1)
                uy1 = uy + jnp.uint32(P1)
                uz1 = uz + jnp.uint32(P2)
                for ci, (cx, cy, cz) in enumerate(CORNERS):
                    h = (ux1 if cx else ux) ^ (uy1 if cy else uy) ^ (uz1 if cz else uz)
                    idx = (h & jnp.uint32(TSZ - 1)).astype(jnp.int32)
                    plsc.store_scatter(grp_vs[par][ci], [rows16],
                                       (idx >> 2) + l * GRP)
                    plsc.store_scatter(off_vs[par][ci], [rows16], (idx & 3) << 1)

        return [
            pltpu.async_copy(tab.at[grp_vs[par][ci]], rows_vs[par][ci], sems[par])
            for ci in range(8)
        ]

    def interp(l, par):
        f0 = 3 * par

        @plsc.parallel_loop(0, C // LANES, unroll=UNROLL)
        def pc(i, l=l, par=par, f0=f0):
            o = i * LANES
            rows16 = o + lane
            fx = plsc.load_gather(frac_v, [jnp.full((LANES,), f0, jnp.int32), rows16])
            fy = plsc.load_gather(frac_v, [jnp.full((LANES,), f0 + 1, jnp.int32), rows16])
            fz = plsc.load_gather(frac_v, [jnp.full((LANES,), f0 + 2, jnp.int32), rows16])
            gx = 1.0 - fx
            gy = 1.0 - fy
            gz = 1.0 - fz
            wxy = {(0, 0): gx * gy, (1, 0): fx * gy,
                   (0, 1): gx * fy, (1, 1): fx * fy}
            acc0 = jnp.zeros((LANES,), jnp.float32)
            acc1 = jnp.zeros((LANES,), jnp.float32)
            for ci, (cx, cy, cz) in enumerate(CORNERS):
                w = wxy[(cx, cy)] * (fz if cz else gz)
                lo = plsc.load_gather(off_vs[par][ci], [rows16])
                v0 = plsc.load_gather(rows_vs[par][ci], [rows16, lo])
                v1 = plsc.load_gather(rows_vs[par][ci], [rows16, lo + 1])
                acc0 = acc0 + v0 * w
                acc1 = acc1 + v1 * w
            plsc.store_scatter(
                obuf_v, [rows16, jnp.full((LANES,), 2 * l, jnp.int32)], acc0)
            plsc.store_scatter(
                obuf_v, [rows16, jnp.full((LANES,), 2 * l + 1, jnp.int32)], acc1)

    def chunk_body(g, carry):
        base = wid * ppt + g * C
        pltpu.sync_copy(xn.at[pl.ds(base, C)], coords_v)

        copies = a_and_fire(0, 0)
        for l in range(N_LEVELS):
            par = l % 2
            nxt = a_and_fire(l + 1, 1 - par) if l + 1 < N_LEVELS else None
            for cp in copies:
                cp.wait()
            interp(l, par)
            copies = nxt

        pltpu.sync_copy(obuf_v, out.at[pl.ds(base, C)])
        return carry

    lax.fori_loop(0, nchunk, chunk_body, 0)


PAIRS_PER_TILE = N_LEVELS * TSZ // NW
PCH = 4096


def _interleave_body(tabp, outp, f0b, f1b, ob, sem):
    wid = lax.axis_index("s") * NC + lax.axis_index("c")
    lane = jnp.arange(LANES, dtype=jnp.int32)
    lvl = wid // 2
    t0 = (wid % 2) * PAIRS_PER_TILE

    def cb(g, carry):
        t = t0 + g * PCH
        f0o = (2 * lvl) * TSZ + t
        pltpu.sync_copy(tabp.at[pl.ds(f0o, PCH)], f0b)
        pltpu.sync_copy(tabp.at[pl.ds(f0o + TSZ, PCH)], f1b)

        @plsc.parallel_loop(0, PCH // LANES, unroll=2)
        def asm(i):
            rows16 = i * LANES + lane
            v0 = plsc.load_gather(f0b, [rows16])
            v1 = plsc.load_gather(f1b, [rows16])
            e0 = rows16 * 2
            plsc.store_scatter(ob, [e0 >> 3, e0 & 7], v0)
            e1 = e0 + 1
            plsc.store_scatter(ob, [e1 >> 3, e1 & 7], v1)

        row0 = (lvl * TSZ + t) >> 2
        pltpu.sync_copy(ob, outp.at[pl.ds(row0, PCH // 4)])
        return carry

    lax.fori_loop(0, PAIRS_PER_TILE // PCH, cb, 0)


def kernel(xc, table, bbox_min, bbox_max):
    n = xc.shape[0]
    assert n % (NW * C) == 0
    scale = jnp.clip(bbox_max - bbox_min, 1e-6, None)
    xn = (xc - bbox_min[None, :]) / scale[None, :]
    tabp = jnp.swapaxes(table, 1, 2).reshape(N_LEVELS * FDIM * TSZ)
    mesh_i = plsc.VectorSubcoreMesh(core_axis_name="c", subcore_axis_name="s")
    fi = pl.kernel(
        _interleave_body,
        out_type=jax.ShapeDtypeStruct((N_LEVELS * GRP, 4 * FDIM), jnp.float32),
        mesh=mesh_i,
        compiler_params=pltpu.CompilerParams(
            needs_layout_passes=False, use_tc_tiling_on_sc=False),
        scratch_types=[
            pltpu.VMEM((PCH,), jnp.float32),
            pltpu.VMEM((PCH,), jnp.float32),
            pltpu.VMEM((PCH // 4, 4 * FDIM), jnp.float32),
            pltpu.SemaphoreType.DMA,
        ],
    )
    tab = fi(tabp)

    mesh = plsc.VectorSubcoreMesh(core_axis_name="c", subcore_axis_name="s")
    f = pl.kernel(
        _body,
        out_type=jax.ShapeDtypeStruct((n, N_LEVELS * FDIM), jnp.float32),
        mesh=mesh,
        compiler_params=pltpu.CompilerParams(
            needs_layout_passes=False, use_tc_tiling_on_sc=False),
        scratch_types=[
            pltpu.VMEM((C, 3), jnp.float32),
            pltpu.VMEM((6, C), jnp.float32),
            *[pltpu.VMEM((C,), jnp.int32) for _ in range(16)],
            *[pltpu.VMEM((C,), jnp.int32) for _ in range(16)],
            *[pltpu.VMEM((C, 4 * FDIM), jnp.float32) for _ in range(16)],
            pltpu.VMEM((C, N_LEVELS * FDIM), jnp.float32),
            pltpu.SemaphoreType.DMA,
            pltpu.SemaphoreType.DMA,
        ],
    )
    return f(xn, tab)
